# baseline (speedup 1.0000x reference)
import jax
import jax.numpy as jnp
from jax.experimental import pallas as pl
from jax.experimental.pallas import tpu as pltpu

_C = 384
_MUL = 1.0 / 40.0
_T_MAX = 2.0
_CH = 8
_N = 64 // _CH
_NBI = 4
_NBO = 4


def _body(t1_ref, t2_ref, out_ref, ibuf, obuf, isem, osem):
    def in_copy(slot, i, k):
        src = (t1_ref, t2_ref)[k]
        return pltpu.make_async_copy(
            src.at[0, pl.ds(i * _CH, _CH)], ibuf.at[slot, k],
            isem.at[slot, k])

    def out_copy(slot, i):
        return pltpu.make_async_copy(
            obuf.at[slot], out_ref.at[pl.ds(i * _CH, _CH)], osem.at[slot])

    def start_in(slot, i):
        in_copy(slot, i, 0).start()
        in_copy(slot, i, 1).start()

    for b in range(min(_NBI, _N)):
        start_in(b, b)

    def step(i, carry):
        si = jax.lax.rem(i, _NBI)
        so = jax.lax.rem(i, _NBO)
        in_copy(si, i, 0).wait()
        in_copy(si, i, 1).wait()

        @pl.when(i >= _NBO)
        def _():
            out_copy(so, i - _NBO).wait()

        a = ibuf[si, 0]
        b = ibuf[si, 1]
        d = ((a[..., :_C] - a[..., _C:]) + (b[..., :_C] - b[..., _C:])) * _MUL
        obuf[so, :, :, :_C] = jnp.minimum(d + _T_MAX, _T_MAX)
        obuf[so, :, :, _C:] = jnp.minimum(_T_MAX - d, _T_MAX)
        out_copy(so, i).start()

        @pl.when(i + _NBI < _N)
        def _():
            start_in(si, i + _NBI)
        return carry

    jax.lax.fori_loop(0, _N, step, 0)

    for b in range(min(_NBO, _N)):
        i = _N - min(_NBO, _N) + b
        out_copy(i % _NBO, i).wait()


def kernel(tj1, tj2):
    t1 = jnp.transpose(tj1, (0, 2, 3, 1))
    t2 = jnp.transpose(tj2, (0, 2, 3, 1))
    out = pl.pallas_call(
        _body,
        in_specs=[pl.BlockSpec(memory_space=pl.ANY),
                  pl.BlockSpec(memory_space=pl.ANY)],
        out_specs=pl.BlockSpec(memory_space=pl.ANY),
        out_shape=jax.ShapeDtypeStruct((64, 64, 2 * _C), jnp.float32),
        scratch_shapes=[
            pltpu.VMEM((_NBI, 2, _CH, 64, 2 * _C), jnp.float32),
            pltpu.VMEM((_NBO, _CH, 64, 2 * _C), jnp.float32),
            pltpu.SemaphoreType.DMA((_NBI, 2)),
            pltpu.SemaphoreType.DMA((_NBO,)),
        ],
    )(t1, t2)
    return jnp.transpose(out, (2, 0, 1))


# speedup vs baseline: 1.0225x; 1.0225x over previous
import jax
import jax.numpy as jnp
from jax.experimental import pallas as pl
from jax.experimental.pallas import tpu as pltpu

_C = 384
_MUL = 1.0 / 40.0
_T_MAX = 2.0
_CH = 2
_N = 64 // _CH
_NBI = 10
_NBO = 10


def _body(t1_ref, t2_ref, out_ref, ibuf, obuf, isem, osem):
    def in_copy(slot, i, k):
        src = (t1_ref, t2_ref)[k]
        return pltpu.make_async_copy(
            src.at[0, pl.ds(i * _CH, _CH)], ibuf.at[slot, k],
            isem.at[slot, k])

    def out_copy(slot, i):
        return pltpu.make_async_copy(
            obuf.at[slot], out_ref.at[pl.ds(i * _CH, _CH)], osem.at[slot])

    def start_in(slot, i):
        in_copy(slot, i, 0).start()
        in_copy(slot, i, 1).start()

    for b in range(min(_NBI, _N)):
        start_in(b, b)

    def step(i, carry):
        si = jax.lax.rem(i, _NBI)
        so = jax.lax.rem(i, _NBO)
        in_copy(si, i, 0).wait()
        in_copy(si, i, 1).wait()

        @pl.when(i >= _NBO)
        def _():
            out_copy(so, i - _NBO).wait()

        a = ibuf[si, 0]
        b = ibuf[si, 1]
        d = ((a[..., :_C] - a[..., _C:]) + (b[..., :_C] - b[..., _C:])) * _MUL
        obuf[so, :, :, :_C] = jnp.minimum(d + _T_MAX, _T_MAX)
        obuf[so, :, :, _C:] = jnp.minimum(_T_MAX - d, _T_MAX)
        out_copy(so, i).start()

        @pl.when(i + _NBI < _N)
        def _():
            start_in(si, i + _NBI)
        return carry

    jax.lax.fori_loop(0, _N, step, 0)

    for b in range(min(_NBO, _N)):
        i = _N - min(_NBO, _N) + b
        out_copy(i % _NBO, i).wait()


def kernel(tj1, tj2):
    t1 = jnp.transpose(tj1, (0, 2, 3, 1))
    t2 = jnp.transpose(tj2, (0, 2, 3, 1))
    out = pl.pallas_call(
        _body,
        in_specs=[pl.BlockSpec(memory_space=pl.ANY),
                  pl.BlockSpec(memory_space=pl.ANY)],
        out_specs=pl.BlockSpec(memory_space=pl.ANY),
        out_shape=jax.ShapeDtypeStruct((64, 64, 2 * _C), jnp.float32),
        scratch_shapes=[
            pltpu.VMEM((_NBI, 2, _CH, 64, 2 * _C), jnp.float32),
            pltpu.VMEM((_NBO, _CH, 64, 2 * _C), jnp.float32),
            pltpu.SemaphoreType.DMA((_NBI, 2)),
            pltpu.SemaphoreType.DMA((_NBO,)),
        ],
    )(t1, t2)
    return jnp.transpose(out, (2, 0, 1))
